# baseline (speedup 1.0000x reference)
import functools

import jax
import jax.numpy as jnp
from jax import lax
from jax.experimental import pallas as pl
from jax.experimental.pallas import tpu as pltpu
from jax.experimental.pallas import tpu_sc as plsc

NC = 2
NS = 16
NW = NC * NS
LANES = 16

_PERM_DNUMS = lax.GatherDimensionNumbers(
    offset_dims=(), collapsed_slice_dims=(0,), start_index_map=(0,))


def _lane_permute(x, idx):
    return lax.gather(
        x, idx[:, None], dimension_numbers=_PERM_DNUMS, slice_sizes=(1,),
        mode=lax.GatherScatterMode.PROMISE_IN_BOUNDS)


@functools.partial(jax.jit, static_argnames=("epw", "chunk"))
def _sc_edge_dot(h, edge_index, *, epw, chunk):
    e_total = edge_index.shape[1]
    d_words = h.shape[1]
    n_chunks = epw // chunk
    assert n_chunks % 2 == 1
    n_pairs = n_chunks // 2
    mesh = plsc.VectorSubcoreMesh(
        core_axis_name="c", subcore_axis_name="s",
        num_cores=NC, num_subcores=NS)

    @functools.partial(
        pl.kernel,
        out_type=jax.ShapeDtypeStruct((1, e_total), jnp.float32),
        mesh=mesh,
        scratch_types=[
            pltpu.VMEM((epw,), jnp.int32),
            pltpu.VMEM((epw,), jnp.int32),
            pltpu.VMEM((chunk, d_words), jnp.int32),
            pltpu.VMEM((chunk, d_words), jnp.int32),
            pltpu.VMEM((chunk, d_words), jnp.int32),
            pltpu.VMEM((chunk, d_words), jnp.int32),
            pltpu.VMEM((epw,), jnp.float32),
            pltpu.SemaphoreType.DMA,
            pltpu.SemaphoreType.DMA,
            pltpu.SemaphoreType.DMA,
            pltpu.SemaphoreType.DMA,
        ],
        compiler_params=pltpu.CompilerParams(
            needs_layout_passes=False, use_tc_tiling_on_sc=False),
    )
    def body(h_hbm, edge_hbm, out_hbm,
             sidx, didx, ub0, vb0, ub1, vb1, sbuf,
             sem_u0, sem_v0, sem_u1, sem_v1):
        wid = lax.axis_index("s") * NC + lax.axis_index("c")
        base = wid * epw
        pltpu.sync_copy(edge_hbm.at[0, pl.ds(base, epw)], sidx)
        pltpu.sync_copy(edge_hbm.at[1, pl.ds(base, epw)], didx)

        lane = lax.iota(jnp.int32, LANES)

        def gather_start(i, ub, vb, sem_u, sem_v):
            off = i * chunk
            pltpu.async_copy(h_hbm.at[sidx.at[pl.ds(off, chunk)]], ub, sem_u)
            pltpu.async_copy(h_hbm.at[didx.at[pl.ds(off, chunk)]], vb, sem_v)

        def gather_wait(i, ub, vb, sem_u, sem_v):
            off = i * chunk
            pltpu.make_async_copy(
                h_hbm.at[sidx.at[pl.ds(off, chunk)]], ub, sem_u).wait()
            pltpu.make_async_copy(
                h_hbm.at[didx.at[pl.ds(off, chunk)]], vb, sem_v).wait()

        def compute(i, ub, vb):
            off = i * chunk

            def group_body(g, c2):
                res = jnp.zeros((LANES,), jnp.float32)
                for j in range(LANES):
                    e = g * LANES + j
                    acc = None
                    for k in range(d_words // LANES):
                        uw = plsc.bitcast(
                            ub[e, pl.ds(k * LANES, LANES)], jnp.bfloat16)
                        vw = plsc.bitcast(
                            vb[e, pl.ds(k * LANES, LANES)], jnp.bfloat16)
                        ua, uo = plsc.unpack(
                            uw, format=plsc.PackFormat.INTERLEAVED,
                            preferred_element_type=jnp.float32)
                        va, vo = plsc.unpack(
                            vw, format=plsc.PackFormat.INTERLEAVED,
                            preferred_element_type=jnp.float32)
                        t = ua * va + uo * vo
                        acc = t if acc is None else acc + t
                    for sh in (8, 4, 2, 1):
                        acc = acc + _lane_permute(acc, lane ^ sh)
                    res = jnp.where(lane == j, acc, res)
                sbuf[pl.ds(off + g * LANES, LANES)] = res
                return c2

            lax.fori_loop(0, chunk // LANES, group_body, 0)

        gather_start(0, ub0, vb0, sem_u0, sem_v0)

        def pair_body(p, carry):
            i0 = 2 * p
            gather_start(i0 + 1, ub1, vb1, sem_u1, sem_v1)
            gather_wait(i0, ub0, vb0, sem_u0, sem_v0)
            compute(i0, ub0, vb0)
            gather_start(i0 + 2, ub0, vb0, sem_u0, sem_v0)
            gather_wait(i0 + 1, ub1, vb1, sem_u1, sem_v1)
            compute(i0 + 1, ub1, vb1)
            return carry

        lax.fori_loop(0, n_pairs, pair_body, 0)
        gather_wait(n_chunks - 1, ub0, vb0, sem_u0, sem_v0)
        compute(n_chunks - 1, ub0, vb0)

        pltpu.sync_copy(sbuf, out_hbm.at[0, pl.ds(base, epw)])

    return body(h, edge_index)


def kernel(h, edge_index):
    e_total = edge_index.shape[1]
    assert e_total % NW == 0
    epw = e_total // NW
    chunk = 80
    assert epw % chunk == 0 and chunk % LANES == 0
    ei = edge_index.astype(jnp.int32)
    lo = lax.bitcast_convert_type(
        h[:, 0::2].astype(jnp.bfloat16), jnp.uint16).astype(jnp.uint32)
    hi = lax.bitcast_convert_type(
        h[:, 1::2].astype(jnp.bfloat16), jnp.uint16).astype(jnp.uint32)
    h_packed = lax.bitcast_convert_type(lo | (hi << 16), jnp.int32)
    score = _sc_edge_dot(h_packed, ei, epw=epw, chunk=chunk)
    return score.reshape(e_total, 1)


# speedup vs baseline: 1.6879x; 1.6879x over previous
import functools

import jax
import jax.numpy as jnp
from jax import lax
from jax.experimental import pallas as pl
from jax.experimental.pallas import tpu as pltpu
from jax.experimental.pallas import tpu_sc as plsc

NC = 2
NS = 16
NW = NC * NS
LANES = 16

_PERM_DNUMS = lax.GatherDimensionNumbers(
    offset_dims=(), collapsed_slice_dims=(0,), start_index_map=(0,))


def _lane_permute(x, idx):
    return lax.gather(
        x, idx[:, None], dimension_numbers=_PERM_DNUMS, slice_sizes=(1,),
        mode=lax.GatherScatterMode.PROMISE_IN_BOUNDS)


@functools.partial(jax.jit, static_argnames=("epw", "chunk"))
def _sc_edge_dot(h, edge_index, *, epw, chunk):
    e_total = edge_index.shape[1]
    d_words = h.shape[1]
    n_chunks = epw // chunk
    assert n_chunks % 2 == 1
    n_pairs = n_chunks // 2
    mesh = plsc.VectorSubcoreMesh(
        core_axis_name="c", subcore_axis_name="s",
        num_cores=NC, num_subcores=NS)

    @functools.partial(
        pl.kernel,
        out_type=jax.ShapeDtypeStruct((1, e_total), jnp.float32),
        mesh=mesh,
        scratch_types=[
            pltpu.VMEM((epw,), jnp.int32),
            pltpu.VMEM((epw,), jnp.int32),
            pltpu.VMEM((chunk, d_words), jnp.int32),
            pltpu.VMEM((chunk, d_words), jnp.int32),
            pltpu.VMEM((chunk, d_words), jnp.int32),
            pltpu.VMEM((chunk, d_words), jnp.int32),
            pltpu.VMEM((epw,), jnp.float32),
            pltpu.SemaphoreType.DMA,
            pltpu.SemaphoreType.DMA,
            pltpu.SemaphoreType.DMA,
            pltpu.SemaphoreType.DMA,
        ],
        compiler_params=pltpu.CompilerParams(
            needs_layout_passes=False, use_tc_tiling_on_sc=False),
    )
    def body(h_hbm, edge_hbm, out_hbm,
             sidx, didx, ub0, vb0, ub1, vb1, sbuf,
             sem_u0, sem_v0, sem_u1, sem_v1):
        wid = lax.axis_index("s") * NC + lax.axis_index("c")
        base = wid * epw
        pltpu.sync_copy(edge_hbm.at[0, pl.ds(base, epw)], sidx)
        pltpu.sync_copy(edge_hbm.at[1, pl.ds(base, epw)], didx)

        lane = lax.iota(jnp.int32, LANES)

        def gather_start(i, ub, vb, sem_u, sem_v):
            off = i * chunk
            pltpu.async_copy(h_hbm.at[sidx.at[pl.ds(off, chunk)]], ub, sem_u)
            pltpu.async_copy(h_hbm.at[didx.at[pl.ds(off, chunk)]], vb, sem_v)

        def gather_wait(i, ub, vb, sem_u, sem_v):
            off = i * chunk
            pltpu.make_async_copy(
                h_hbm.at[sidx.at[pl.ds(off, chunk)]], ub, sem_u).wait()
            pltpu.make_async_copy(
                h_hbm.at[didx.at[pl.ds(off, chunk)]], vb, sem_v).wait()

        def compute(i, ub, vb):
            off = i * chunk

            def group_body(g, c2):
                res = jnp.zeros((LANES,), jnp.float32)
                for j in range(LANES):
                    e = g * LANES + j
                    acc = None
                    for k in range(d_words // LANES):
                        uw = plsc.bitcast(
                            ub[e, pl.ds(k * LANES, LANES)], jnp.bfloat16)
                        vw = plsc.bitcast(
                            vb[e, pl.ds(k * LANES, LANES)], jnp.bfloat16)
                        ua, uo = plsc.unpack(
                            uw, format=plsc.PackFormat.INTERLEAVED,
                            preferred_element_type=jnp.float32)
                        va, vo = plsc.unpack(
                            vw, format=plsc.PackFormat.INTERLEAVED,
                            preferred_element_type=jnp.float32)
                        t = ua * va + uo * vo
                        acc = t if acc is None else acc + t
                    for sh in (8, 4, 2, 1):
                        acc = acc + _lane_permute(acc, lane ^ sh)
                    res = jnp.where(lane == j, acc, res)
                sbuf[pl.ds(off + g * LANES, LANES)] = res
                return c2

            lax.fori_loop(0, chunk // LANES, group_body, 0)

        gather_start(0, ub0, vb0, sem_u0, sem_v0)

        def pair_body(p, carry):
            i0 = 2 * p
            gather_start(i0 + 1, ub1, vb1, sem_u1, sem_v1)
            gather_wait(i0, ub0, vb0, sem_u0, sem_v0)
            compute(i0, ub0, vb0)
            gather_start(i0 + 2, ub0, vb0, sem_u0, sem_v0)
            gather_wait(i0 + 1, ub1, vb1, sem_u1, sem_v1)
            compute(i0 + 1, ub1, vb1)
            return carry

        lax.fori_loop(0, n_pairs, pair_body, 0)
        gather_wait(n_chunks - 1, ub0, vb0, sem_u0, sem_v0)
        compute(n_chunks - 1, ub0, vb0)

        pltpu.sync_copy(sbuf, out_hbm.at[0, pl.ds(base, epw)])

    return body(h, edge_index)


def kernel(h, edge_index):
    e_total = edge_index.shape[1]
    assert e_total % NW == 0
    epw = e_total // NW
    chunk = 80
    assert epw % chunk == 0 and chunk % LANES == 0
    ei = (edge_index if edge_index.dtype == jnp.int32
          else edge_index.astype(jnp.int32))
    d_half = h.shape[1] // 2
    lo = lax.bitcast_convert_type(
        h[:, :d_half].astype(jnp.bfloat16), jnp.uint16).astype(jnp.uint32)
    hi = lax.bitcast_convert_type(
        h[:, d_half:].astype(jnp.bfloat16), jnp.uint16).astype(jnp.uint32)
    h_packed = lax.bitcast_convert_type(lo | (hi << 16), jnp.int32)
    score = _sc_edge_dot(h_packed, ei, epw=epw, chunk=chunk)
    return score.reshape(e_total, 1)


# speedup vs baseline: 1.8137x; 1.0745x over previous
import functools

import jax
import jax.numpy as jnp
from jax import lax
from jax.experimental import pallas as pl
from jax.experimental.pallas import tpu as pltpu
from jax.experimental.pallas import tpu_sc as plsc

NC = 2
NS = 16
NW = NC * NS
LANES = 16

_PERM_DNUMS = lax.GatherDimensionNumbers(
    offset_dims=(), collapsed_slice_dims=(0,), start_index_map=(0,))


def _lane_permute(x, idx):
    return lax.gather(
        x, idx[:, None], dimension_numbers=_PERM_DNUMS, slice_sizes=(1,),
        mode=lax.GatherScatterMode.PROMISE_IN_BOUNDS)


@functools.partial(jax.jit, static_argnames=("epw", "chunk"))
def _sc_edge_dot(h, edge_index, *, epw, chunk):
    e_total = edge_index.shape[1]
    d_words = h.shape[1]
    n_chunks = epw // chunk
    assert n_chunks % 2 == 1
    n_pairs = n_chunks // 2
    mesh = plsc.VectorSubcoreMesh(
        core_axis_name="c", subcore_axis_name="s",
        num_cores=NC, num_subcores=NS)

    @functools.partial(
        pl.kernel,
        out_type=jax.ShapeDtypeStruct((1, e_total), jnp.float32),
        mesh=mesh,
        scratch_types=[
            pltpu.VMEM((epw,), jnp.int32),
            pltpu.VMEM((epw,), jnp.int32),
            pltpu.VMEM((chunk, d_words), jnp.int32),
            pltpu.VMEM((chunk, d_words), jnp.int32),
            pltpu.VMEM((chunk, d_words), jnp.int32),
            pltpu.VMEM((chunk, d_words), jnp.int32),
            pltpu.VMEM((epw,), jnp.float32),
            pltpu.SemaphoreType.DMA,
            pltpu.SemaphoreType.DMA,
            pltpu.SemaphoreType.DMA,
            pltpu.SemaphoreType.DMA,
        ],
        compiler_params=pltpu.CompilerParams(
            needs_layout_passes=False, use_tc_tiling_on_sc=False),
    )
    def body(h_hbm, edge_hbm, out_hbm,
             sidx, didx, ub0, vb0, ub1, vb1, sbuf,
             sem_u0, sem_v0, sem_u1, sem_v1):
        wid = lax.axis_index("s") * NC + lax.axis_index("c")
        base = wid * epw
        pltpu.sync_copy(edge_hbm.at[0, pl.ds(base, epw)], sidx)
        pltpu.sync_copy(edge_hbm.at[1, pl.ds(base, epw)], didx)

        lane = lax.iota(jnp.int32, LANES)

        def gather_start(i, ub, vb, sem_u, sem_v):
            off = i * chunk
            pltpu.async_copy(h_hbm.at[sidx.at[pl.ds(off, chunk)]], ub, sem_u)
            pltpu.async_copy(h_hbm.at[didx.at[pl.ds(off, chunk)]], vb, sem_v)

        def gather_wait(i, ub, vb, sem_u, sem_v):
            off = i * chunk
            pltpu.make_async_copy(
                h_hbm.at[sidx.at[pl.ds(off, chunk)]], ub, sem_u).wait()
            pltpu.make_async_copy(
                h_hbm.at[didx.at[pl.ds(off, chunk)]], vb, sem_v).wait()

        def compute(i, ub, vb):
            off = i * chunk

            def group_body(g, c2):
                res = jnp.zeros((LANES,), jnp.float32)
                for j in range(LANES):
                    e = g * LANES + j
                    acc = None
                    for k in range(d_words // LANES):
                        uw = plsc.bitcast(
                            ub[e, pl.ds(k * LANES, LANES)], jnp.bfloat16)
                        vw = plsc.bitcast(
                            vb[e, pl.ds(k * LANES, LANES)], jnp.bfloat16)
                        pa, po = plsc.unpack(
                            uw * vw, format=plsc.PackFormat.INTERLEAVED,
                            preferred_element_type=jnp.float32)
                        t = pa + po
                        acc = t if acc is None else acc + t
                    res = jnp.where(lane == j, jnp.sum(acc), res)
                sbuf[pl.ds(off + g * LANES, LANES)] = res
                return c2

            lax.fori_loop(0, chunk // LANES, group_body, 0)

        gather_start(0, ub0, vb0, sem_u0, sem_v0)

        def pair_body(p, carry):
            i0 = 2 * p
            gather_start(i0 + 1, ub1, vb1, sem_u1, sem_v1)
            gather_wait(i0, ub0, vb0, sem_u0, sem_v0)
            compute(i0, ub0, vb0)
            gather_start(i0 + 2, ub0, vb0, sem_u0, sem_v0)
            gather_wait(i0 + 1, ub1, vb1, sem_u1, sem_v1)
            compute(i0 + 1, ub1, vb1)
            return carry

        lax.fori_loop(0, n_pairs, pair_body, 0)
        gather_wait(n_chunks - 1, ub0, vb0, sem_u0, sem_v0)
        compute(n_chunks - 1, ub0, vb0)

        pltpu.sync_copy(sbuf, out_hbm.at[0, pl.ds(base, epw)])

    return body(h, edge_index)


def kernel(h, edge_index):
    e_total = edge_index.shape[1]
    assert e_total % NW == 0
    epw = e_total // NW
    chunk = 80
    assert epw % chunk == 0 and chunk % LANES == 0
    ei = (edge_index if edge_index.dtype == jnp.int32
          else edge_index.astype(jnp.int32))
    d_half = h.shape[1] // 2
    lo = lax.bitcast_convert_type(
        h[:, :d_half].astype(jnp.bfloat16), jnp.uint16).astype(jnp.uint32)
    hi = lax.bitcast_convert_type(
        h[:, d_half:].astype(jnp.bfloat16), jnp.uint16).astype(jnp.uint32)
    h_packed = lax.bitcast_convert_type(lo | (hi << 16), jnp.int32)
    score = _sc_edge_dot(h_packed, ei, epw=epw, chunk=chunk)
    return score.reshape(e_total, 1)


# speedup vs baseline: 2.2720x; 1.2527x over previous
import functools

import jax
import jax.numpy as jnp
from jax import lax
from jax.experimental import pallas as pl
from jax.experimental.pallas import tpu as pltpu
from jax.experimental.pallas import tpu_sc as plsc

NC = 2
NS = 16
NW = NC * NS
LANES = 16

_PERM_DNUMS = lax.GatherDimensionNumbers(
    offset_dims=(), collapsed_slice_dims=(0,), start_index_map=(0,))


def _lane_permute(x, idx):
    return lax.gather(
        x, idx[:, None], dimension_numbers=_PERM_DNUMS, slice_sizes=(1,),
        mode=lax.GatherScatterMode.PROMISE_IN_BOUNDS)


@functools.partial(jax.jit, static_argnames=("epw", "chunk"))
def _sc_edge_dot(h, edge_index, *, epw, chunk):
    e_total = edge_index.shape[1]
    n_nodes = h.shape[0]
    d_words = h.shape[1]
    n_chunks = epw // chunk
    n_pairs = n_chunks // 2
    rows_per_tile = n_nodes // NS
    mesh = plsc.VectorSubcoreMesh(
        core_axis_name="c", subcore_axis_name="s",
        num_cores=NC, num_subcores=NS)

    @functools.partial(
        pl.kernel,
        out_type=jax.ShapeDtypeStruct((1, e_total), jnp.float32),
        mesh=mesh,
        scratch_types=[
            pltpu.VMEM_SHARED((n_nodes, d_words), jnp.int32),
            pltpu.VMEM((epw,), jnp.int32),
            pltpu.VMEM((epw,), jnp.int32),
            pltpu.VMEM((chunk, d_words), jnp.int32),
            pltpu.VMEM((chunk, d_words), jnp.int32),
            pltpu.VMEM((chunk, d_words), jnp.int32),
            pltpu.VMEM((chunk, d_words), jnp.int32),
            pltpu.VMEM((epw,), jnp.float32),
            pltpu.SemaphoreType.DMA,
            pltpu.SemaphoreType.DMA,
            pltpu.SemaphoreType.DMA,
            pltpu.SemaphoreType.DMA,
        ],
        compiler_params=pltpu.CompilerParams(
            needs_layout_passes=False, use_tc_tiling_on_sc=False),
    )
    def body(h_hbm, edge_hbm, out_hbm,
             h_sp, sidx, didx, ub0, vb0, ub1, vb1, sbuf,
             sem_u0, sem_v0, sem_u1, sem_v1):
        sid = lax.axis_index("s")
        wid = sid * NC + lax.axis_index("c")
        base = wid * epw
        row0 = sid * rows_per_tile
        pltpu.sync_copy(h_hbm.at[pl.ds(row0, rows_per_tile)],
                        h_sp.at[pl.ds(row0, rows_per_tile)])
        pltpu.sync_copy(edge_hbm.at[0, pl.ds(base, epw)], sidx)
        pltpu.sync_copy(edge_hbm.at[1, pl.ds(base, epw)], didx)
        plsc.subcore_barrier()

        lane = lax.iota(jnp.int32, LANES)

        def gather_start(i, ub, vb, sem_u, sem_v):
            off = i * chunk
            pltpu.async_copy(h_sp.at[sidx.at[pl.ds(off, chunk)]], ub, sem_u)
            pltpu.async_copy(h_sp.at[didx.at[pl.ds(off, chunk)]], vb, sem_v)

        def gather_wait(i, ub, vb, sem_u, sem_v):
            off = i * chunk
            pltpu.make_async_copy(
                h_sp.at[sidx.at[pl.ds(off, chunk)]], ub, sem_u).wait()
            pltpu.make_async_copy(
                h_sp.at[didx.at[pl.ds(off, chunk)]], vb, sem_v).wait()

        def compute(i, ub, vb):
            off = i * chunk

            def group_body(g, c2):
                res = jnp.zeros((LANES,), jnp.float32)
                for j in range(LANES):
                    e = g * LANES + j
                    acc = None
                    for k in range(d_words // LANES):
                        uw = plsc.bitcast(
                            ub[e, pl.ds(k * LANES, LANES)], jnp.bfloat16)
                        vw = plsc.bitcast(
                            vb[e, pl.ds(k * LANES, LANES)], jnp.bfloat16)
                        pa, po = plsc.unpack(
                            uw * vw, format=plsc.PackFormat.INTERLEAVED,
                            preferred_element_type=jnp.float32)
                        t = pa + po
                        acc = t if acc is None else acc + t
                    res = jnp.where(lane == j, jnp.sum(acc), res)
                sbuf[pl.ds(off + g * LANES, LANES)] = res
                return c2

            lax.fori_loop(0, chunk // LANES, group_body, 0)

        gather_start(0, ub0, vb0, sem_u0, sem_v0)

        def pair_body(p, carry):
            i0 = 2 * p
            gather_start(i0 + 1, ub1, vb1, sem_u1, sem_v1)
            gather_wait(i0, ub0, vb0, sem_u0, sem_v0)
            compute(i0, ub0, vb0)

            @pl.when(i0 + 2 < n_chunks)
            def _():
                gather_start(i0 + 2, ub0, vb0, sem_u0, sem_v0)

            gather_wait(i0 + 1, ub1, vb1, sem_u1, sem_v1)
            compute(i0 + 1, ub1, vb1)
            return carry

        lax.fori_loop(0, n_pairs, pair_body, 0)
        if n_chunks % 2 == 1:
            gather_wait(n_chunks - 1, ub0, vb0, sem_u0, sem_v0)
            compute(n_chunks - 1, ub0, vb0)

        pltpu.sync_copy(sbuf, out_hbm.at[0, pl.ds(base, epw)])

    return body(h, edge_index)


def kernel(h, edge_index):
    e_total = edge_index.shape[1]
    assert e_total % NW == 0 and h.shape[0] % NS == 0
    epw = e_total // NW
    chunk = 80
    assert epw % chunk == 0 and chunk % LANES == 0
    ei = (edge_index if edge_index.dtype == jnp.int32
          else edge_index.astype(jnp.int32))
    d_half = h.shape[1] // 2
    lo = lax.bitcast_convert_type(
        h[:, :d_half].astype(jnp.bfloat16), jnp.uint16).astype(jnp.uint32)
    hi = lax.bitcast_convert_type(
        h[:, d_half:].astype(jnp.bfloat16), jnp.uint16).astype(jnp.uint32)
    h_packed = lax.bitcast_convert_type(lo | (hi << 16), jnp.int32)
    score = _sc_edge_dot(h_packed, ei, epw=epw, chunk=chunk)
    return score.reshape(e_total, 1)


# speedup vs baseline: 2.2901x; 1.0080x over previous
import functools

import jax
import jax.numpy as jnp
from jax import lax
from jax.experimental import pallas as pl
from jax.experimental.pallas import tpu as pltpu
from jax.experimental.pallas import tpu_sc as plsc

NC = 2
NS = 16
NW = NC * NS
LANES = 16

_PERM_DNUMS = lax.GatherDimensionNumbers(
    offset_dims=(), collapsed_slice_dims=(0,), start_index_map=(0,))


def _lane_permute(x, idx):
    return lax.gather(
        x, idx[:, None], dimension_numbers=_PERM_DNUMS, slice_sizes=(1,),
        mode=lax.GatherScatterMode.PROMISE_IN_BOUNDS)


@functools.partial(jax.jit, static_argnames=("epw", "chunk"))
def _sc_edge_dot(h, edge_index, *, epw, chunk):
    e_total = NW * epw
    n_nodes = h.shape[0]
    d_words = h.shape[1]
    n_chunks = epw // chunk
    n_pairs = n_chunks // 2
    rows_per_tile = n_nodes // NS
    mesh = plsc.VectorSubcoreMesh(
        core_axis_name="c", subcore_axis_name="s",
        num_cores=NC, num_subcores=NS)

    @functools.partial(
        pl.kernel,
        out_type=jax.ShapeDtypeStruct((1, e_total), jnp.float32),
        mesh=mesh,
        scratch_types=[
            pltpu.VMEM_SHARED((n_nodes, d_words), jnp.int32),
            pltpu.VMEM((n_chunks, 2 * chunk), jnp.int32),
            pltpu.VMEM((2 * chunk, d_words), jnp.int32),
            pltpu.VMEM((2 * chunk, d_words), jnp.int32),
            pltpu.VMEM((epw,), jnp.float32),
            pltpu.SemaphoreType.DMA,
            pltpu.SemaphoreType.DMA,
        ],
        compiler_params=pltpu.CompilerParams(
            needs_layout_passes=False, use_tc_tiling_on_sc=False),
    )
    def body(h_hbm, edge_hbm, out_hbm,
             h_sp, eidx, buf0, buf1, sbuf, sem0, sem1):
        sid = lax.axis_index("s")
        wid = sid * NC + lax.axis_index("c")
        base = wid * epw
        row0 = sid * rows_per_tile
        pltpu.sync_copy(h_hbm.at[pl.ds(row0, rows_per_tile)],
                        h_sp.at[pl.ds(row0, rows_per_tile)])
        pltpu.sync_copy(edge_hbm.at[0, wid], eidx.at[:, pl.ds(0, chunk)])
        pltpu.sync_copy(edge_hbm.at[1, wid], eidx.at[:, pl.ds(chunk, chunk)])
        plsc.subcore_barrier()

        lane = lax.iota(jnp.int32, LANES)

        def gather_start(i, buf, sem):
            pltpu.async_copy(h_sp.at[eidx.at[i]], buf, sem)

        def gather_wait(i, buf, sem):
            pltpu.make_async_copy(h_sp.at[eidx.at[i]], buf, sem).wait()

        def compute(i, buf):
            off = i * chunk

            def group_body(g, c2):
                res = jnp.zeros((LANES,), jnp.float32)
                for j in range(LANES):
                    e = g * LANES + j
                    acc = None
                    for k in range(d_words // LANES):
                        uw = plsc.bitcast(
                            buf[e, pl.ds(k * LANES, LANES)], jnp.bfloat16)
                        vw = plsc.bitcast(
                            buf[chunk + e, pl.ds(k * LANES, LANES)],
                            jnp.bfloat16)
                        pa, po = plsc.unpack(
                            uw * vw, format=plsc.PackFormat.INTERLEAVED,
                            preferred_element_type=jnp.float32)
                        t = pa + po
                        acc = t if acc is None else acc + t
                    res = jnp.where(lane == j, jnp.sum(acc), res)
                sbuf[pl.ds(off + g * LANES, LANES)] = res
                return c2

            lax.fori_loop(0, chunk // LANES, group_body, 0)

        gather_start(0, buf0, sem0)

        def pair_body(p, carry):
            i0 = 2 * p
            gather_start(i0 + 1, buf1, sem1)
            gather_wait(i0, buf0, sem0)
            compute(i0, buf0)

            @pl.when(i0 + 2 < n_chunks)
            def _():
                gather_start(i0 + 2, buf0, sem0)

            gather_wait(i0 + 1, buf1, sem1)
            compute(i0 + 1, buf1)
            return carry

        lax.fori_loop(0, n_pairs, pair_body, 0)
        if n_chunks % 2 == 1:
            gather_wait(n_chunks - 1, buf0, sem0)
            compute(n_chunks - 1, buf0)

        pltpu.sync_copy(sbuf, out_hbm.at[0, pl.ds(base, epw)])

    return body(h, edge_index)


def kernel(h, edge_index):
    e_total = edge_index.shape[1]
    assert e_total % NW == 0 and h.shape[0] % NS == 0
    epw = e_total // NW
    chunk = 80
    assert epw % chunk == 0 and chunk % LANES == 0
    ei = (edge_index if edge_index.dtype == jnp.int32
          else edge_index.astype(jnp.int32))
    d_half = h.shape[1] // 2
    lo = lax.bitcast_convert_type(
        h[:, :d_half].astype(jnp.bfloat16), jnp.uint16).astype(jnp.uint32)
    hi = lax.bitcast_convert_type(
        h[:, d_half:].astype(jnp.bfloat16), jnp.uint16).astype(jnp.uint32)
    h_packed = lax.bitcast_convert_type(lo | (hi << 16), jnp.int32)
    ei4 = ei.reshape(2, NW, epw // chunk, chunk)
    score = _sc_edge_dot(h_packed, ei4, epw=epw, chunk=chunk)
    return score.reshape(e_total, 1)
